# baseline (speedup 1.0000x reference)
import functools

import jax
import jax.numpy as jnp
from jax import lax
from jax.experimental import pallas as pl
from jax.experimental.pallas import tpu as pltpu
from jax.experimental.pallas import tpu_sc as plsc

RES = 128
MARGIN = 0.1
NC = 2
NS = 16
NW = NC * NS
C = 1024
G = 128


def _bounds_body(p_ref, o_ref):
    p = p_ref[...]
    lb = jnp.min(p, axis=1, keepdims=True)
    ub = jnp.max(p, axis=1, keepdims=True)
    ext = ub - lb
    lower = lb - MARGIN * ext
    upper = ub + MARGIN * ext
    fac = (RES - 1.0) / (upper - lower)
    o_ref[...] = jnp.concatenate([lower, fac], axis=1)


def _sqrt_body(x_ref, o_ref):
    o_ref[...] = jnp.sqrt(x_ref[...])


def _sc_body(q_hbm, grid_hbm, px_hbm, py_hbm, pz_hbm, consts_hbm,
             d2_out, nn_out,
             q_v, xs, ys, zs, idx_v, nn_v, pxg, pyg, pzg, d_v, consts_v,
             sem1, sem2):
    per_w = d2_out.shape[0] // NW
    nchunk = per_w // C
    wid = lax.axis_index("s") * NC + lax.axis_index("c")
    base = wid * per_w

    pltpu.sync_copy(consts_hbm, consts_v)
    cv = consts_v[...]
    l0 = cv[0]
    l1 = cv[1]
    l2 = cv[2]
    f0 = cv[3]
    f1 = cv[4]
    f2 = cv[5]
    inv_s = cv[6]
    s = cv[7]
    lane = lax.iota(jnp.int32, 16)

    def quant(v, l, f):
        t = (v * inv_s - l) * f
        t = jnp.minimum(jnp.maximum(t, 0.0), float(RES - 1))
        return (t + 0.5).astype(jnp.int32)

    def chunk_body(ci, _):
        qbase = base + ci * C
        pltpu.sync_copy(q_hbm.at[pl.ds(qbase * 3, C * 3)], q_v)

        def grp(j, _):
            qi = j * 16 + lane
            bi = qi * 3
            x = plsc.load_gather(q_v, [bi])
            y = plsc.load_gather(q_v, [bi + 1])
            z = plsc.load_gather(q_v, [bi + 2])
            sl = pl.ds(j * 16, 16)
            xs[sl] = x
            ys[sl] = y
            zs[sl] = z
            gx = quant(x, l0, f0)
            gy = quant(y, l1, f1)
            gz = quant(z, l2, f2)
            idx_v[sl] = (gx * RES + gy) * RES + gz
            return 0

        lax.fori_loop(0, C // 16, grp, 0, unroll=8)

        descs = [
            pltpu.async_copy(grid_hbm.at[idx_v.at[pl.ds(k * G, G)]],
                             nn_v.at[pl.ds(k * G, G)], sem1)
            for k in range(C // G)
        ]
        for d in descs:
            d.wait()
        descs2 = [
            pltpu.async_copy(src.at[nn_v.at[pl.ds(k * G, G)]],
                             dst.at[pl.ds(k * G, G)], sem2)
            for k in range(C // G)
            for src, dst in ((px_hbm, pxg), (py_hbm, pyg), (pz_hbm, pzg))
        ]
        for d in descs2:
            d.wait()

        def grp2(j, _):
            sl = pl.ds(j * 16, 16)
            dx = pxg[sl] * s - xs[sl]
            dy = pyg[sl] * s - ys[sl]
            dz = pzg[sl] * s - zs[sl]
            d_v[sl] = dx * dx + dy * dy + dz * dz
            return 0

        lax.fori_loop(0, C // 16, grp2, 0, unroll=8)

        pltpu.sync_copy(d_v, d2_out.at[pl.ds(qbase, C)])
        pltpu.sync_copy(nn_v, nn_out.at[pl.ds(qbase, C)])
        return 0

    lax.fori_loop(0, nchunk, chunk_body, 0)


def kernel(query_points, scale, points, nn_idxs_grid):
    nq = query_points.shape[0]
    qpad = ((nq + NW * C - 1) // (NW * C)) * (NW * C)

    pts_t = points.T
    bounds = pl.pallas_call(
        _bounds_body,
        out_shape=jax.ShapeDtypeStruct((3, 2), jnp.float32),
    )(pts_t)
    inv_s = 1.0 / scale
    consts = jnp.concatenate(
        [bounds[:, 0], bounds[:, 1],
         jnp.stack([inv_s, scale]), jnp.zeros((8,), jnp.float32)])

    q_flat = jnp.pad(query_points, ((0, qpad - nq), (0, 0))).reshape(-1)
    grid_flat = nn_idxs_grid.reshape(-1)
    px, py, pz = pts_t[0], pts_t[1], pts_t[2]

    mesh = plsc.VectorSubcoreMesh(core_axis_name="c", subcore_axis_name="s",
                                  num_cores=NC, num_subcores=NS)
    sc = pl.kernel(
        _sc_body,
        out_type=(jax.ShapeDtypeStruct((qpad,), jnp.float32),
                  jax.ShapeDtypeStruct((qpad,), jnp.int32)),
        mesh=mesh,
        compiler_params=pltpu.CompilerParams(needs_layout_passes=False),
        scratch_types=[
            pltpu.VMEM((3 * C,), jnp.float32),
            pltpu.VMEM((C,), jnp.float32),
            pltpu.VMEM((C,), jnp.float32),
            pltpu.VMEM((C,), jnp.float32),
            pltpu.VMEM((C,), jnp.int32),
            pltpu.VMEM((C,), jnp.int32),
            pltpu.VMEM((C,), jnp.float32),
            pltpu.VMEM((C,), jnp.float32),
            pltpu.VMEM((C,), jnp.float32),
            pltpu.VMEM((C,), jnp.float32),
            pltpu.VMEM((16,), jnp.float32),
            pltpu.SemaphoreType.DMA,
            pltpu.SemaphoreType.DMA,
        ],
    )
    d2, nn = sc(q_flat, grid_flat, px, py, pz, consts)

    dist = pl.pallas_call(
        _sqrt_body,
        out_shape=jax.ShapeDtypeStruct((qpad // 128, 128), jnp.float32),
    )(d2.reshape(qpad // 128, 128)).reshape(-1)

    return (dist[:nq], nn[:nq])


# speedup vs baseline: 1.1078x; 1.1078x over previous
import jax
import jax.numpy as jnp
from jax import lax
from jax.experimental import pallas as pl
from jax.experimental.pallas import tpu as pltpu
from jax.experimental.pallas import tpu_sc as plsc

RES = 128
MARGIN = 0.1
NC = 2
NS = 16
NW = NC * NS
C = 1024
G = 128


def _minmax_body(p_ref, o_ref):
    p = p_ref[...]
    o_ref[...] = jnp.concatenate(
        [jnp.min(p, axis=0, keepdims=True),
         jnp.max(p, axis=0, keepdims=True)], axis=0)


def _sqrt_body(x_ref, o_ref):
    o_ref[...] = jnp.sqrt(x_ref[...])


def _sc_body(q_hbm, grid_hbm, pts_hbm, consts_hbm, d2_out, nn_out,
             q_v, xs, ys, zs, idx_v, nn_v, ia_v, ib_v, ic_v,
             pxg, pyg, pzg, d_v, consts_v, sem1, sem2):
    nq = d2_out.shape[0]
    work = (nq // (16 * NW)) * 16
    nfull = (work // C)
    wid = lax.axis_index("s") * NC + lax.axis_index("c")
    base = wid * work

    pltpu.sync_copy(consts_hbm, consts_v)
    cv = consts_v[...]
    l0 = cv[0]
    l1 = cv[1]
    l2 = cv[2]
    f0 = cv[3]
    f1 = cv[4]
    f2 = cv[5]
    inv_s = cv[6]
    s = cv[7]
    lane = lax.iota(jnp.int32, 16)

    def quant(v, l, f):
        t = (v * inv_s - l) * f
        t = jnp.minimum(jnp.maximum(t, 0.0), float(RES - 1))
        return (t + 0.5).astype(jnp.int32)

    def do_chunk(qbase):
        pltpu.sync_copy(q_hbm.at[pl.ds(qbase * 3, C * 3)], q_v)

        def grp(j, _):
            bi = (j * 16 + lane) * 3
            x = plsc.load_gather(q_v, [bi])
            y = plsc.load_gather(q_v, [bi + 1])
            z = plsc.load_gather(q_v, [bi + 2])
            sl = pl.ds(j * 16, 16)
            xs[sl] = x
            ys[sl] = y
            zs[sl] = z
            gx = quant(x, l0, f0)
            gy = quant(y, l1, f1)
            gz = quant(z, l2, f2)
            idx_v[sl] = (gx * RES + gy) * RES + gz
            return 0

        lax.fori_loop(0, C // 16, grp, 0, unroll=8)

        descs = [
            pltpu.async_copy(grid_hbm.at[idx_v.at[pl.ds(k * G, G)]],
                             nn_v.at[pl.ds(k * G, G)], sem1)
            for k in range(C // G)
        ]
        for d in descs:
            d.wait()

        def grp1b(j, _):
            sl = pl.ds(j * 16, 16)
            b = nn_v[sl] * 3
            ia_v[sl] = b
            ib_v[sl] = b + 1
            ic_v[sl] = b + 2
            return 0

        lax.fori_loop(0, C // 16, grp1b, 0, unroll=8)

        descs2 = [
            pltpu.async_copy(pts_hbm.at[src.at[pl.ds(k * G, G)]],
                             dst.at[pl.ds(k * G, G)], sem2)
            for k in range(C // G)
            for src, dst in ((ia_v, pxg), (ib_v, pyg), (ic_v, pzg))
        ]
        for d in descs2:
            d.wait()

        def grp2(j, _):
            sl = pl.ds(j * 16, 16)
            dx = pxg[sl] * s - xs[sl]
            dy = pyg[sl] * s - ys[sl]
            dz = pzg[sl] * s - zs[sl]
            d_v[sl] = dx * dx + dy * dy + dz * dz
            return 0

        lax.fori_loop(0, C // 16, grp2, 0, unroll=8)

        pltpu.sync_copy(d_v, d2_out.at[pl.ds(qbase, C)])
        pltpu.sync_copy(nn_v, nn_out.at[pl.ds(qbase, C)])

    def chunk_body(ci, _):
        do_chunk(base + ci * C)
        return 0

    lax.fori_loop(0, nfull, chunk_body, 0)
    end = jnp.where(wid == NW - 1, nq, base + work)
    do_chunk(end - C)


def kernel(query_points, scale, points, nn_idxs_grid):
    nq = query_points.shape[0]
    npts = points.shape[0]

    mm = pl.pallas_call(
        _minmax_body,
        out_shape=jax.ShapeDtypeStruct((2, 120), jnp.float32),
    )(points.reshape(npts // 40, 120))
    lb = jnp.min(mm[0].reshape(40, 3), axis=0)
    ub = jnp.max(mm[1].reshape(40, 3), axis=0)
    ext = ub - lb
    lower = lb - MARGIN * ext
    upper = ub + MARGIN * ext
    fac = (RES - 1.0) / (upper - lower)
    inv_s = 1.0 / scale
    consts = jnp.concatenate(
        [lower, fac, jnp.stack([inv_s, scale]), jnp.zeros((8,), jnp.float32)])

    q_flat = query_points.reshape(-1)
    grid_flat = nn_idxs_grid.reshape(-1)
    pts_flat = points.reshape(-1)

    mesh = plsc.VectorSubcoreMesh(core_axis_name="c", subcore_axis_name="s",
                                  num_cores=NC, num_subcores=NS)
    sc = pl.kernel(
        _sc_body,
        out_type=(jax.ShapeDtypeStruct((nq,), jnp.float32),
                  jax.ShapeDtypeStruct((nq,), jnp.int32)),
        mesh=mesh,
        compiler_params=pltpu.CompilerParams(needs_layout_passes=False),
        scratch_types=[
            pltpu.VMEM((3 * C,), jnp.float32),
            pltpu.VMEM((C,), jnp.float32),
            pltpu.VMEM((C,), jnp.float32),
            pltpu.VMEM((C,), jnp.float32),
            pltpu.VMEM((C,), jnp.int32),
            pltpu.VMEM((C,), jnp.int32),
            pltpu.VMEM((C,), jnp.int32),
            pltpu.VMEM((C,), jnp.int32),
            pltpu.VMEM((C,), jnp.int32),
            pltpu.VMEM((C,), jnp.float32),
            pltpu.VMEM((C,), jnp.float32),
            pltpu.VMEM((C,), jnp.float32),
            pltpu.VMEM((C,), jnp.float32),
            pltpu.VMEM((16,), jnp.float32),
            pltpu.SemaphoreType.DMA,
            pltpu.SemaphoreType.DMA,
        ],
    )
    d2, nn = sc(q_flat, grid_flat, pts_flat, consts)

    dist = pl.pallas_call(
        _sqrt_body,
        out_shape=jax.ShapeDtypeStruct((1000, nq // 1000), jnp.float32),
    )(d2.reshape(1000, nq // 1000)).reshape(-1)

    return (dist, nn)


# speedup vs baseline: 9.9877x; 9.0156x over previous
import jax
import jax.numpy as jnp
from jax import lax
from jax.experimental import pallas as pl
from jax.experimental.pallas import tpu as pltpu
from jax.experimental.pallas import tpu_sc as plsc

RES = 128
MARGIN = 0.1
NC = 2
NS = 16
NW = NC * NS
C = 1024
G = 128


def _minmax_body(px_ref, py_ref, pz_ref, o_ref):
    mins = jnp.stack([jnp.min(px_ref[...]), jnp.min(py_ref[...]),
                      jnp.min(pz_ref[...])])
    maxs = jnp.stack([jnp.max(px_ref[...]), jnp.max(py_ref[...]),
                      jnp.max(pz_ref[...])])
    o_ref[...] = jnp.stack([mins, maxs])


def _sqrt_body(x_ref, o_ref):
    o_ref[...] = jnp.sqrt(x_ref[...])


def _sc_body(qx_hbm, qy_hbm, qz_hbm, grid_hbm, px_hbm, py_hbm, pz_hbm,
             consts_hbm, d2_out, nn_out,
             xs, ys, zs, idx_v, nn_v, pxg, pyg, pzg, d_v, consts_v,
             sem1, sem2):
    nq = d2_out.shape[0]
    work = (nq // (16 * NW)) * 16
    nfull = (work // C)
    wid = lax.axis_index("s") * NC + lax.axis_index("c")
    base = wid * work

    pltpu.sync_copy(consts_hbm, consts_v)
    cv = consts_v[...]
    l0 = cv[0]
    l1 = cv[1]
    l2 = cv[2]
    f0 = cv[3]
    f1 = cv[4]
    f2 = cv[5]
    inv_s = cv[6]
    s = cv[7]

    def quant(v, l, f):
        t = (v * inv_s - l) * f
        t = jnp.minimum(jnp.maximum(t, 0.0), float(RES - 1))
        return (t + 0.5).astype(jnp.int32)

    def do_chunk(qbase):
        src_sl = pl.ds(qbase, C)
        pltpu.sync_copy(qx_hbm.at[src_sl], xs)
        pltpu.sync_copy(qy_hbm.at[src_sl], ys)
        pltpu.sync_copy(qz_hbm.at[src_sl], zs)

        def grp(j, _):
            sl = pl.ds(j * 16, 16)
            gx = quant(xs[sl], l0, f0)
            gy = quant(ys[sl], l1, f1)
            gz = quant(zs[sl], l2, f2)
            idx_v[sl] = (gx * RES + gy) * RES + gz
            return 0

        lax.fori_loop(0, C // 16, grp, 0, unroll=8)

        descs = [
            pltpu.async_copy(grid_hbm.at[idx_v.at[pl.ds(k * G, G)]],
                             nn_v.at[pl.ds(k * G, G)], sem1)
            for k in range(C // G)
        ]
        for d in descs:
            d.wait()

        descs2 = [
            pltpu.async_copy(src.at[nn_v.at[pl.ds(k * G, G)]],
                             dst.at[pl.ds(k * G, G)], sem2)
            for k in range(C // G)
            for src, dst in ((px_hbm, pxg), (py_hbm, pyg), (pz_hbm, pzg))
        ]
        for d in descs2:
            d.wait()

        def grp2(j, _):
            sl = pl.ds(j * 16, 16)
            dx = pxg[sl] * s - xs[sl]
            dy = pyg[sl] * s - ys[sl]
            dz = pzg[sl] * s - zs[sl]
            d_v[sl] = dx * dx + dy * dy + dz * dz
            return 0

        lax.fori_loop(0, C // 16, grp2, 0, unroll=8)

        pltpu.sync_copy(d_v, d2_out.at[pl.ds(qbase, C)])
        pltpu.sync_copy(nn_v, nn_out.at[pl.ds(qbase, C)])

    def chunk_body(ci, _):
        do_chunk(base + ci * C)
        return 0

    lax.fori_loop(0, nfull, chunk_body, 0)
    end = jnp.where(wid == NW - 1, nq, base + work)
    do_chunk(end - C)


def kernel(query_points, scale, points, nn_idxs_grid):
    nq = query_points.shape[0]

    qx = query_points[:, 0]
    qy = query_points[:, 1]
    qz = query_points[:, 2]
    px = points[:, 0]
    py = points[:, 1]
    pz = points[:, 2]
    grid_flat = nn_idxs_grid.reshape(-1)

    mm = pl.pallas_call(
        _minmax_body,
        out_shape=jax.ShapeDtypeStruct((2, 3), jnp.float32),
    )(px, py, pz)
    lb = mm[0]
    ub = mm[1]
    ext = ub - lb
    lower = lb - MARGIN * ext
    upper = ub + MARGIN * ext
    fac = (RES - 1.0) / (upper - lower)
    inv_s = 1.0 / scale
    consts = jnp.concatenate(
        [lower, fac, jnp.stack([inv_s, scale]), jnp.zeros((8,), jnp.float32)])

    mesh = plsc.VectorSubcoreMesh(core_axis_name="c", subcore_axis_name="s",
                                  num_cores=NC, num_subcores=NS)
    sc = pl.kernel(
        _sc_body,
        out_type=(jax.ShapeDtypeStruct((nq,), jnp.float32),
                  jax.ShapeDtypeStruct((nq,), jnp.int32)),
        mesh=mesh,
        compiler_params=pltpu.CompilerParams(needs_layout_passes=False),
        scratch_types=[
            pltpu.VMEM((C,), jnp.float32),
            pltpu.VMEM((C,), jnp.float32),
            pltpu.VMEM((C,), jnp.float32),
            pltpu.VMEM((C,), jnp.int32),
            pltpu.VMEM((C,), jnp.int32),
            pltpu.VMEM((C,), jnp.float32),
            pltpu.VMEM((C,), jnp.float32),
            pltpu.VMEM((C,), jnp.float32),
            pltpu.VMEM((C,), jnp.float32),
            pltpu.VMEM((16,), jnp.float32),
            pltpu.SemaphoreType.DMA,
            pltpu.SemaphoreType.DMA,
        ],
    )
    d2, nn = sc(qx, qy, qz, grid_flat, px, py, pz, consts)

    dist = pl.pallas_call(
        _sqrt_body,
        out_shape=jax.ShapeDtypeStruct((1000, nq // 1000), jnp.float32),
    )(d2.reshape(1000, nq // 1000)).reshape(-1)

    return (dist, nn)


# speedup vs baseline: 14.2681x; 1.4286x over previous
import jax
import jax.numpy as jnp
from jax import lax
from jax.experimental import pallas as pl
from jax.experimental.pallas import tpu as pltpu
from jax.experimental.pallas import tpu_sc as plsc

RES = 128
MARGIN = 0.1
NC = 2
NS = 16
NW = NC * NS
C = 1024
G = 128


def _minmax_body(px_ref, py_ref, pz_ref, o_ref):
    mins = jnp.stack([jnp.min(px_ref[...]), jnp.min(py_ref[...]),
                      jnp.min(pz_ref[...])])
    maxs = jnp.stack([jnp.max(px_ref[...]), jnp.max(py_ref[...]),
                      jnp.max(pz_ref[...])])
    o_ref[...] = jnp.stack([mins, maxs])


def _sc_body(qx_hbm, qy_hbm, qz_hbm, grid_hbm, px_hbm, py_hbm, pz_hbm,
             consts_hbm, dist_out, nn_out, *sc):
    buf = (dict(xs=sc[0], ys=sc[1], zs=sc[2], idx=sc[3], nn=sc[4],
                pxg=sc[5], pyg=sc[6], pzg=sc[7], d=sc[8]),
           dict(xs=sc[9], ys=sc[10], zs=sc[11], idx=sc[12], nn=sc[13],
                pxg=sc[14], pyg=sc[15], pzg=sc[16], d=sc[17]))
    consts_v = sc[18]
    sem_q = (sc[19], sc[20])
    sem_g = (sc[21], sc[22])
    sem_p = (sc[23], sc[24])
    sem_w = (sc[25], sc[26])

    nq = dist_out.shape[0]
    work = (nq // (16 * NW)) * 16
    nfull = work // C
    m = nfull + 1
    wid = lax.axis_index("s") * NC + lax.axis_index("c")
    base = wid * work
    end = jnp.where(wid == NW - 1, nq, base + work)

    def start(ci):
        return jnp.minimum(base + ci * C, end - C)

    pltpu.sync_copy(consts_hbm, consts_v)
    cv = consts_v[...]
    l0, l1, l2 = cv[0], cv[1], cv[2]
    f0, f1, f2 = cv[3], cv[4], cv[5]
    inv_s, s = cv[6], cv[7]

    def quant(v, l, f):
        t = (v * inv_s - l) * f
        t = jnp.minimum(jnp.maximum(t, 0.0), float(RES - 1))
        return (t + 0.5).astype(jnp.int32)

    def stage_a(ci, p):
        b = buf[p]
        sl = pl.ds(start(ci), C)
        pltpu.async_copy(qx_hbm.at[sl], b["xs"], sem_q[p])
        pltpu.async_copy(qy_hbm.at[sl], b["ys"], sem_q[p])
        pltpu.async_copy(qz_hbm.at[sl], b["zs"], sem_q[p])

    def stage_b(ci, p, drain_w):
        b = buf[p]
        if drain_w:
            osl = pl.ds(start(ci - 2), C)
            pltpu.make_async_copy(b["d"], dist_out.at[osl], sem_w[p]).wait()
            pltpu.make_async_copy(b["nn"], nn_out.at[osl], sem_w[p]).wait()
        sl = pl.ds(start(ci), C)
        pltpu.make_async_copy(qx_hbm.at[sl], b["xs"], sem_q[p]).wait()
        pltpu.make_async_copy(qy_hbm.at[sl], b["ys"], sem_q[p]).wait()
        pltpu.make_async_copy(qz_hbm.at[sl], b["zs"], sem_q[p]).wait()

        def grp(j, _):
            jsl = pl.ds(j * 16, 16)
            gx = quant(b["xs"][jsl], l0, f0)
            gy = quant(b["ys"][jsl], l1, f1)
            gz = quant(b["zs"][jsl], l2, f2)
            b["idx"][jsl] = (gx * RES + gy) * RES + gz
            return 0

        lax.fori_loop(0, C // 16, grp, 0, unroll=8)
        for k in range(C // G):
            ksl = pl.ds(k * G, G)
            pltpu.async_copy(grid_hbm.at[b["idx"].at[ksl]],
                             b["nn"].at[ksl], sem_g[p])

    def stage_c(ci, p):
        b = buf[p]
        for k in range(C // G):
            ksl = pl.ds(k * G, G)
            pltpu.make_async_copy(grid_hbm.at[b["idx"].at[ksl]],
                                  b["nn"].at[ksl], sem_g[p]).wait()
        for k in range(C // G):
            ksl = pl.ds(k * G, G)
            pltpu.async_copy(px_hbm.at[b["nn"].at[ksl]],
                             b["pxg"].at[ksl], sem_p[p])
            pltpu.async_copy(py_hbm.at[b["nn"].at[ksl]],
                             b["pyg"].at[ksl], sem_p[p])
            pltpu.async_copy(pz_hbm.at[b["nn"].at[ksl]],
                             b["pzg"].at[ksl], sem_p[p])

    def stage_d(ci, p):
        b = buf[p]
        for k in range(C // G):
            ksl = pl.ds(k * G, G)
            pltpu.make_async_copy(px_hbm.at[b["nn"].at[ksl]],
                                  b["pxg"].at[ksl], sem_p[p]).wait()
            pltpu.make_async_copy(py_hbm.at[b["nn"].at[ksl]],
                                  b["pyg"].at[ksl], sem_p[p]).wait()
            pltpu.make_async_copy(pz_hbm.at[b["nn"].at[ksl]],
                                  b["pzg"].at[ksl], sem_p[p]).wait()

        def grp2(j, _):
            jsl = pl.ds(j * 16, 16)
            dx = b["pxg"][jsl] * s - b["xs"][jsl]
            dy = b["pyg"][jsl] * s - b["ys"][jsl]
            dz = b["pzg"][jsl] * s - b["zs"][jsl]
            d2 = dx * dx + dy * dy + dz * dz
            yi = jnp.int32(0x5F3759DF) - lax.shift_right_logical(
                plsc.bitcast(d2, jnp.int32), 1)
            y = plsc.bitcast(yi, jnp.float32)
            hd = 0.5 * d2
            y = y * (1.5 - hd * y * y)
            y = y * (1.5 - hd * y * y)
            b["d"][jsl] = d2 * y
            return 0

        lax.fori_loop(0, C // 16, grp2, 0, unroll=8)
        osl = pl.ds(start(ci), C)
        pltpu.async_copy(b["d"], dist_out.at[osl], sem_w[p])
        pltpu.async_copy(b["nn"], nn_out.at[osl], sem_w[p])

    stage_a(0, 0)
    stage_a(1, 1)
    stage_b(0, 0, False)

    def pair_steps(i0, first):
        stage_c(i0, 0)
        stage_b(i0 + 1, 1, not first)
        stage_d(i0, 0)
        stage_a(i0 + 2, 0)
        stage_c(i0 + 1, 1)
        stage_b(i0 + 2, 0, True)
        stage_d(i0 + 1, 1)
        stage_a(i0 + 3, 1)

    pair_steps(0, True)

    def pair_body(pair, _):
        pair_steps(2 * pair, False)
        return 0

    lax.fori_loop(1, (m - 1) // 2, pair_body, 0)
    stage_c(m - 1, 0)
    stage_d(m - 1, 0)
    sl = pl.ds(start(m), C)
    pltpu.make_async_copy(qx_hbm.at[sl], buf[1]["xs"], sem_q[1]).wait()
    pltpu.make_async_copy(qy_hbm.at[sl], buf[1]["ys"], sem_q[1]).wait()
    pltpu.make_async_copy(qz_hbm.at[sl], buf[1]["zs"], sem_q[1]).wait()
    osl0 = pl.ds(start(m - 1), C)
    pltpu.make_async_copy(buf[0]["d"], dist_out.at[osl0], sem_w[0]).wait()
    pltpu.make_async_copy(buf[0]["nn"], nn_out.at[osl0], sem_w[0]).wait()
    osl1 = pl.ds(start(m - 2), C)
    pltpu.make_async_copy(buf[1]["d"], dist_out.at[osl1], sem_w[1]).wait()
    pltpu.make_async_copy(buf[1]["nn"], nn_out.at[osl1], sem_w[1]).wait()


def kernel(query_points, scale, points, nn_idxs_grid):
    nq = query_points.shape[0]

    qx = query_points[:, 0]
    qy = query_points[:, 1]
    qz = query_points[:, 2]
    px = points[:, 0]
    py = points[:, 1]
    pz = points[:, 2]
    grid_flat = nn_idxs_grid.reshape(-1)

    mm = pl.pallas_call(
        _minmax_body,
        out_shape=jax.ShapeDtypeStruct((2, 3), jnp.float32),
    )(px, py, pz)
    lb = mm[0]
    ub = mm[1]
    ext = ub - lb
    lower = lb - MARGIN * ext
    upper = ub + MARGIN * ext
    fac = (RES - 1.0) / (upper - lower)
    inv_s = 1.0 / scale
    consts = jnp.concatenate(
        [lower, fac, jnp.stack([inv_s, scale]), jnp.zeros((8,), jnp.float32)])

    mesh = plsc.VectorSubcoreMesh(core_axis_name="c", subcore_axis_name="s",
                                  num_cores=NC, num_subcores=NS)
    fbuf = [pltpu.VMEM((C,), jnp.float32)] * 3 \
        + [pltpu.VMEM((C,), jnp.int32)] * 2 \
        + [pltpu.VMEM((C,), jnp.float32)] * 4
    sc = pl.kernel(
        _sc_body,
        out_type=(jax.ShapeDtypeStruct((nq,), jnp.float32),
                  jax.ShapeDtypeStruct((nq,), jnp.int32)),
        mesh=mesh,
        compiler_params=pltpu.CompilerParams(needs_layout_passes=False),
        scratch_types=fbuf + fbuf + [pltpu.VMEM((16,), jnp.float32)]
        + [pltpu.SemaphoreType.DMA] * 8,
    )
    dist, nn = sc(qx, qy, qz, grid_flat, px, py, pz, consts)
    return (dist, nn)
